# baseline (speedup 1.0000x reference)
import functools

import jax
import jax.numpy as jnp
from jax import lax
from jax.experimental import pallas as pl
from jax.experimental.pallas import tpu as pltpu
from jax.experimental.pallas import tpu_sc as plsc

B = 16384
F = 26
V = 100000
D = 32
C = 13
H1 = 128
H2 = 64
NCLS = 2

XB = 2048
NXB = 49
VPAD = NXB * XB
FB = (F + 3) // 4

G = 28
NW = 32
PER_W = (B * G) // NW
IDX_ROW = 128
ROWS_PER_W = PER_W // IDX_ROW
ROWS_PER_CHUNK = 4
NCHUNK = ROWS_PER_W // ROWS_PER_CHUNK
CHUNK = ROWS_PER_CHUNK * IDX_ROW


def _tr_body(in_ref, out_ref):
    h = in_ref[...].reshape(4 * D, XB)
    out_ref[...] = h.T


@jax.jit
def _tc_transpose(table_t):
    return pl.pallas_call(
        _tr_body,
        grid=(FB, NXB),
        in_specs=[pl.BlockSpec((4, D, XB), lambda fb, b: (fb, 0, b))],
        out_specs=pl.BlockSpec((XB, 128),
                               lambda fb, b: (fb * NXB + b, 0)),
        out_shape=jax.ShapeDtypeStruct((FB * NXB * XB, 128), jnp.float32),
    )(table_t)


def _gather_body(table_hbm, idx_hbm, out_hbm, idx_v, rows_v, sem):
    c = lax.axis_index("c")
    s = lax.axis_index("s")
    wid = s * 2 + c
    pltpu.sync_copy(idx_hbm.at[pl.ds(wid * ROWS_PER_W, ROWS_PER_W)], idx_v)
    base = wid * PER_W

    def chunk_body(ci, _):
        r0 = ci * ROWS_PER_CHUNK
        for j in range(ROWS_PER_CHUNK):
            pltpu.async_copy(
                table_hbm.at[idx_v.at[r0 + j]],
                rows_v.at[pl.ds(j * IDX_ROW, IDX_ROW)],
                sem,
            )
        for j in range(ROWS_PER_CHUNK):
            pltpu.make_async_copy(
                table_hbm.at[idx_v.at[r0 + j]],
                rows_v.at[pl.ds(j * IDX_ROW, IDX_ROW)],
                sem,
            ).wait()
        pltpu.sync_copy(rows_v, out_hbm.at[pl.ds(base + ci * CHUNK, CHUNK)])
        return 0

    lax.fori_loop(0, NCHUNK, chunk_body, 0)


@jax.jit
def _sc_gather(table_rows, idx2d):
    mesh = plsc.VectorSubcoreMesh(core_axis_name="c", subcore_axis_name="s")
    return pl.kernel(
        _gather_body,
        out_type=jax.ShapeDtypeStruct((B * G, D), jnp.float32),
        mesh=mesh,
        scratch_types=[
            pltpu.VMEM((ROWS_PER_W, IDX_ROW), jnp.int32),
            pltpu.VMEM((CHUNK, D), jnp.float32),
            pltpu.SemaphoreType.DMA,
        ],
        compiler_params=pltpu.CompilerParams(use_tc_tiling_on_sc=False),
    )(table_rows, idx2d)


def _mlp_body(emb_ref, xct_ref, w1a_ref, w1b_ref, b1_ref, w2_ref, b2_ref,
              w3_ref, b3_ref, out_ref):
    h = jnp.dot(emb_ref[...], w1a_ref[...], preferred_element_type=jnp.float32)
    h = h + lax.dot_general(xct_ref[...], w1b_ref[...],
                            (((0,), (0,)), ((), ())),
                            preferred_element_type=jnp.float32)
    h = jnp.maximum(h + b1_ref[...], 0.0)
    h = jnp.dot(h, w2_ref[...], preferred_element_type=jnp.float32)
    h = jnp.maximum(h + b2_ref[...], 0.0)
    o = jnp.dot(h, w3_ref[...], preferred_element_type=jnp.float32)
    out_ref[...] = o + b3_ref[...]


BM = 1024


@jax.jit
def _mlp(emb, xct, w1a, w1b, b1, w2p, b2p, w3p, b3p):
    grid = (B // BM,)
    return pl.pallas_call(
        _mlp_body,
        grid=grid,
        in_specs=[
            pl.BlockSpec((BM, G * D), lambda i: (i, 0)),
            pl.BlockSpec((C, BM), lambda i: (0, i)),
            pl.BlockSpec((G * D, H1), lambda i: (0, 0)),
            pl.BlockSpec((C, H1), lambda i: (0, 0)),
            pl.BlockSpec((1, H1), lambda i: (0, 0)),
            pl.BlockSpec((H1, 128), lambda i: (0, 0)),
            pl.BlockSpec((1, 128), lambda i: (0, 0)),
            pl.BlockSpec((128, 128), lambda i: (0, 0)),
            pl.BlockSpec((1, 128), lambda i: (0, 0)),
        ],
        out_specs=pl.BlockSpec((BM, 128), lambda i: (i, 0)),
        out_shape=jax.ShapeDtypeStruct((B, 128), jnp.float32),
    )(emb, xct, w1a, w1b, b1, w2p, b2p, w3p, b3p)


def kernel(x_cat, x_cont, tables, W1, b1, W2, b2, W3, b3):
    table_t = tables.transpose(0, 2, 1)
    table_rows = _tc_transpose(table_t).reshape(FB * NXB * XB * 4, D)

    x = x_cat.astype(jnp.int32)
    farr = jnp.arange(F, dtype=jnp.int32)
    fblk = (farr // 4 * (NXB * XB * 4))[None, :]
    fk = (farr % 4)[None, :]
    flat_idx = fblk + (x // XB) * (XB * 4) + (x % XB) * 4 + fk
    flat28 = jnp.pad(flat_idx, ((0, 0), (0, G - F)))
    idx2d = flat28.reshape((B * G) // IDX_ROW, IDX_ROW)

    emb = _sc_gather(table_rows, idx2d).reshape(B, G * D)

    xct = x_cont.T
    w1a = jnp.pad(W1[:F * D], ((0, (G - F) * D), (0, 0)))
    w1b = W1[F * D:]
    w2p = jnp.pad(W2, ((0, 0), (0, 128 - H2)))
    b2p = jnp.pad(b2, (0, 128 - H2)).reshape(1, 128)
    w3p = jnp.pad(W3, ((0, 128 - H2), (0, 128 - NCLS)))
    b3p = jnp.pad(b3, (0, 128 - NCLS)).reshape(1, 128)

    out = _mlp(emb, xct, w1a, w1b, b1.reshape(1, H1), w2p, b2p, w3p, b3p)
    return out[:, :NCLS]


# speedup vs baseline: 1.6750x; 1.6750x over previous
import functools

import jax
import jax.numpy as jnp
from jax import lax
from jax.experimental import pallas as pl
from jax.experimental.pallas import tpu as pltpu
from jax.experimental.pallas import tpu_sc as plsc

B = 16384
F = 26
V = 100000
D = 32
C = 13
H1 = 128
H2 = 64
NCLS = 2

XB = 2048
NXB = 49
VPAD = NXB * XB
FB = (F + 3) // 4

G = 28
NW = 32
PER_W = (B * G) // NW
IDX_ROW = 128
ROWS_PER_W = PER_W // IDX_ROW
ROWS_PER_CHUNK = 4
NCHUNK = ROWS_PER_W // ROWS_PER_CHUNK
CHUNK = ROWS_PER_CHUNK * IDX_ROW


def _tr_body(in_ref, out_ref):
    h = in_ref[...].reshape(4 * D, XB)
    out_ref[...] = h.T


@jax.jit
def _tc_transpose(table_t):
    return pl.pallas_call(
        _tr_body,
        grid=(FB, NXB),
        in_specs=[pl.BlockSpec((4, D, XB), lambda fb, b: (fb, 0, b))],
        out_specs=pl.BlockSpec((XB, 128),
                               lambda fb, b: (fb * NXB + b, 0)),
        out_shape=jax.ShapeDtypeStruct((FB * NXB * XB, 128), jnp.float32),
    )(table_t)


def _gather_body(table_hbm, idx_hbm, out_hbm, idx_v, rows_v, sem):
    c = lax.axis_index("c")
    s = lax.axis_index("s")
    wid = s * 2 + c
    pltpu.sync_copy(idx_hbm.at[pl.ds(wid * ROWS_PER_W, ROWS_PER_W)], idx_v)
    base = wid * PER_W

    def chunk_body(ci, _):
        r0 = ci * ROWS_PER_CHUNK
        for j in range(ROWS_PER_CHUNK):
            pltpu.async_copy(
                table_hbm.at[idx_v.at[r0 + j]],
                rows_v.at[pl.ds(j * IDX_ROW, IDX_ROW)],
                sem,
            )
        for j in range(ROWS_PER_CHUNK):
            pltpu.make_async_copy(
                table_hbm.at[idx_v.at[r0 + j]],
                rows_v.at[pl.ds(j * IDX_ROW, IDX_ROW)],
                sem,
            ).wait()
        pltpu.sync_copy(rows_v, out_hbm.at[pl.ds(base + ci * CHUNK, CHUNK)])
        return 0

    lax.fori_loop(0, NCHUNK, chunk_body, 0)


@jax.jit
def _sc_gather(table_rows, idx2d):
    mesh = plsc.VectorSubcoreMesh(core_axis_name="c", subcore_axis_name="s")
    return pl.kernel(
        _gather_body,
        out_type=jax.ShapeDtypeStruct((B * G, D), jnp.float32),
        mesh=mesh,
        scratch_types=[
            pltpu.VMEM((ROWS_PER_W, IDX_ROW), jnp.int32),
            pltpu.VMEM((CHUNK, D), jnp.float32),
            pltpu.SemaphoreType.DMA,
        ],
        compiler_params=pltpu.CompilerParams(use_tc_tiling_on_sc=False),
    )(table_rows, idx2d)


def _mlp_body(emb_ref, xct_ref, w1a_ref, w1b_ref, b1_ref, w2_ref, b2_ref,
              w3_ref, b3_ref, out_ref):
    h = jnp.dot(emb_ref[...], w1a_ref[...], preferred_element_type=jnp.float32)
    h = h + lax.dot_general(xct_ref[...], w1b_ref[...],
                            (((0,), (0,)), ((), ())),
                            preferred_element_type=jnp.float32)
    h = jnp.maximum(h + b1_ref[...], 0.0)
    h = jnp.dot(h, w2_ref[...], preferred_element_type=jnp.float32)
    h = jnp.maximum(h + b2_ref[...], 0.0)
    o = jnp.dot(h, w3_ref[...], preferred_element_type=jnp.float32)
    out_ref[...] = o + b3_ref[...]


BM = 1024


@jax.jit
def _mlp(emb, xct, w1a, w1b, b1, w2p, b2p, w3p, b3p):
    grid = (B // BM,)
    return pl.pallas_call(
        _mlp_body,
        grid=grid,
        in_specs=[
            pl.BlockSpec((BM, G * D), lambda i: (i, 0)),
            pl.BlockSpec((C, BM), lambda i: (0, i)),
            pl.BlockSpec((G * D, H1), lambda i: (0, 0)),
            pl.BlockSpec((C, H1), lambda i: (0, 0)),
            pl.BlockSpec((1, H1), lambda i: (0, 0)),
            pl.BlockSpec((H1, 128), lambda i: (0, 0)),
            pl.BlockSpec((1, 128), lambda i: (0, 0)),
            pl.BlockSpec((128, 128), lambda i: (0, 0)),
            pl.BlockSpec((1, 128), lambda i: (0, 0)),
        ],
        out_specs=pl.BlockSpec((BM, 128), lambda i: (i, 0)),
        out_shape=jax.ShapeDtypeStruct((B, 128), jnp.float32),
    )(emb, xct, w1a, w1b, b1, w2p, b2p, w3p, b3p)


def kernel(x_cat, x_cont, tables, W1, b1, W2, b2, W3, b3):
    table_t = tables.transpose(0, 2, 1)
    table_rows = _tc_transpose(table_t).reshape(FB * NXB * XB * 4, D)

    x = x_cat.astype(jnp.int32)
    farr = jnp.arange(F, dtype=jnp.int32)
    fblk = (farr // 4 * (NXB * XB * 4))[None, :]
    fk = (farr % 4)[None, :]
    flat_idx = fblk + (x // XB) * (XB * 4) + (x % XB) * 4 + fk
    dummy = (jnp.arange(B, dtype=jnp.int32)[:, None] * (G - F)
             + jnp.arange(G - F, dtype=jnp.int32)[None, :])
    flat28 = jnp.concatenate([flat_idx, dummy], axis=1)
    idx2d = flat28.reshape((B * G) // IDX_ROW, IDX_ROW)

    emb = _sc_gather(table_rows, idx2d).reshape(B, G * D)

    xct = x_cont.T
    w1a = jnp.pad(W1[:F * D], ((0, (G - F) * D), (0, 0)))
    w1b = W1[F * D:]
    w2p = jnp.pad(W2, ((0, 0), (0, 128 - H2)))
    b2p = jnp.pad(b2, (0, 128 - H2)).reshape(1, 128)
    w3p = jnp.pad(W3, ((0, 128 - H2), (0, 128 - NCLS)))
    b3p = jnp.pad(b3, (0, 128 - NCLS)).reshape(1, 128)

    out = _mlp(emb, xct, w1a, w1b, b1.reshape(1, H1), w2p, b2p, w3p, b3p)
    return out[:, :NCLS]


# speedup vs baseline: 1.6940x; 1.0113x over previous
import functools

import jax
import jax.numpy as jnp
from jax import lax
from jax.experimental import pallas as pl
from jax.experimental.pallas import tpu as pltpu
from jax.experimental.pallas import tpu_sc as plsc

B = 16384
F = 26
V = 100000
D = 32
C = 13
H1 = 128
H2 = 64
NCLS = 2

XB = 2048
NXB = 49
VPAD = NXB * XB
FB = (F + 3) // 4
SLAB = NXB * XB * 4

NW = 32
PER_W = (B * 4) // NW
IDX_ROW = 128
ROWS_PER_W = PER_W // IDX_ROW
ROWS_PER_CHUNK = 4
NCHUNK = ROWS_PER_W // ROWS_PER_CHUNK
CHUNK = ROWS_PER_CHUNK * IDX_ROW


def _tr_body(in_ref, out_ref):
    h = in_ref[...].reshape(4 * D, XB)
    out_ref[...] = h.T


def _tc_transpose_group(table_t, fb):
    return pl.pallas_call(
        _tr_body,
        grid=(NXB,),
        in_specs=[pl.BlockSpec((4, D, XB), lambda b: (fb, 0, b))],
        out_specs=pl.BlockSpec((XB, 128), lambda b: (b, 0)),
        out_shape=jax.ShapeDtypeStruct((NXB * XB, 128), jnp.float32),
        name=f"transpose_g{fb}",
    )(table_t)


def _gather_body(table_hbm, idx_hbm, out_hbm, idx_v, rows_v, sem):
    c = lax.axis_index("c")
    s = lax.axis_index("s")
    wid = s * 2 + c
    pltpu.sync_copy(idx_hbm.at[pl.ds(wid * ROWS_PER_W, ROWS_PER_W)], idx_v)
    base = wid * PER_W

    def chunk_body(ci, _):
        r0 = ci * ROWS_PER_CHUNK
        for j in range(ROWS_PER_CHUNK):
            pltpu.async_copy(
                table_hbm.at[idx_v.at[r0 + j]],
                rows_v.at[pl.ds(j * IDX_ROW, IDX_ROW)],
                sem,
            )
        for j in range(ROWS_PER_CHUNK):
            pltpu.make_async_copy(
                table_hbm.at[idx_v.at[r0 + j]],
                rows_v.at[pl.ds(j * IDX_ROW, IDX_ROW)],
                sem,
            ).wait()
        pltpu.sync_copy(rows_v, out_hbm.at[pl.ds(base + ci * CHUNK, CHUNK)])
        return 0

    lax.fori_loop(0, NCHUNK, chunk_body, 0)


def _sc_gather(table_rows, idx2d):
    mesh = plsc.VectorSubcoreMesh(core_axis_name="c", subcore_axis_name="s")
    return pl.kernel(
        _gather_body,
        out_type=jax.ShapeDtypeStruct((B * 4, D), jnp.float32),
        mesh=mesh,
        scratch_types=[
            pltpu.VMEM((ROWS_PER_W, IDX_ROW), jnp.int32),
            pltpu.VMEM((CHUNK, D), jnp.float32),
            pltpu.SemaphoreType.DMA,
        ],
        compiler_params=pltpu.CompilerParams(use_tc_tiling_on_sc=False),
    )(table_rows, idx2d)


def _mlp_body(e0, e1, e2, e3, e4, e5, e6, xct_ref, w0, w1, w2, w3, w4, w5,
              w6, w1b_ref, b1_ref, w2_ref, b2_ref, w3_ref, b3_ref, out_ref):
    embs = (e0, e1, e2, e3, e4, e5, e6)
    ws = (w0, w1, w2, w3, w4, w5, w6)
    h = jnp.dot(embs[0][...], ws[0][...], preferred_element_type=jnp.float32)
    for i in range(1, FB):
        h = h + jnp.dot(embs[i][...], ws[i][...],
                        preferred_element_type=jnp.float32)
    h = h + lax.dot_general(xct_ref[...], w1b_ref[...],
                            (((0,), (0,)), ((), ())),
                            preferred_element_type=jnp.float32)
    h = jnp.maximum(h + b1_ref[...], 0.0)
    h = jnp.dot(h, w2_ref[...], preferred_element_type=jnp.float32)
    h = jnp.maximum(h + b2_ref[...], 0.0)
    o = jnp.dot(h, w3_ref[...], preferred_element_type=jnp.float32)
    out_ref[...] = o + b3_ref[...]


BM = 1024


def _mlp(embs, xct, w1g, w1b, b1, w2p, b2p, w3p, b3p):
    grid = (B // BM,)
    emb_specs = [pl.BlockSpec((BM, 128), lambda i: (i, 0)) for _ in range(FB)]
    w_specs = [pl.BlockSpec((128, H1), lambda i: (0, 0)) for _ in range(FB)]
    return pl.pallas_call(
        _mlp_body,
        grid=grid,
        in_specs=emb_specs + [pl.BlockSpec((C, BM), lambda i: (0, i))]
        + w_specs + [
            pl.BlockSpec((C, H1), lambda i: (0, 0)),
            pl.BlockSpec((1, H1), lambda i: (0, 0)),
            pl.BlockSpec((H1, 128), lambda i: (0, 0)),
            pl.BlockSpec((1, 128), lambda i: (0, 0)),
            pl.BlockSpec((128, 128), lambda i: (0, 0)),
            pl.BlockSpec((1, 128), lambda i: (0, 0)),
        ],
        out_specs=pl.BlockSpec((BM, 128), lambda i: (i, 0)),
        out_shape=jax.ShapeDtypeStruct((B, 128), jnp.float32),
    )(*embs, xct, *w1g, w1b, b1, w2p, b2p, w3p, b3p)


@jax.jit
def kernel(x_cat, x_cont, tables, W1, b1, W2, b2, W3, b3):
    table_t = tables.transpose(0, 2, 1)

    x = x_cat.astype(jnp.int32)
    fk = (jnp.arange(F, dtype=jnp.int32) % 4)[None, :]
    loc_idx = (x // XB) * (XB * 4) + (x % XB) * 4 + fk
    dummy = (jnp.arange(B, dtype=jnp.int32)[:, None] * 2
             + jnp.arange(2, dtype=jnp.int32)[None, :])
    loc28 = jnp.concatenate([loc_idx, dummy], axis=1)

    embs = []
    for fb in range(FB):
        slab = _tc_transpose_group(table_t, fb)
        rows = slab.reshape(SLAB, D)
        idx2d = loc28[:, fb * 4:(fb + 1) * 4].reshape((B * 4) // IDX_ROW,
                                                      IDX_ROW)
        embs.append(_sc_gather(rows, idx2d).reshape(B, 128))

    xct = x_cont.T
    w1a = jnp.pad(W1[:F * D], ((0, (FB * 4 - F) * D), (0, 0)))
    w1g = [w1a[fb * 128:(fb + 1) * 128] for fb in range(FB)]
    w1b = W1[F * D:]
    w2p = jnp.pad(W2, ((0, 0), (0, 128 - H2)))
    b2p = jnp.pad(b2, (0, 128 - H2)).reshape(1, 128)
    w3p = jnp.pad(W3, ((0, 128 - H2), (0, 128 - NCLS)))
    b3p = jnp.pad(b3, (0, 128 - NCLS)).reshape(1, 128)

    out = _mlp(embs, xct, w1g, w1b, b1.reshape(1, H1), w2p, b2p, w3p, b3p)
    return out[:, :NCLS]


# speedup vs baseline: 1.7488x; 1.0324x over previous
import functools

import jax
import jax.numpy as jnp
from jax import lax
from jax.experimental import pallas as pl
from jax.experimental.pallas import tpu as pltpu
from jax.experimental.pallas import tpu_sc as plsc

B = 16384
F = 26
V = 100000
D = 32
C = 13
H1 = 128
H2 = 64
NCLS = 2

XB = 2048
NXB = 49
VPAD = NXB * XB
FB = (F + 3) // 4
SLAB = NXB * XB * 4

NW = 32
PER_W = (B * 4) // NW
IDX_ROW = 128
ROWS_PER_W = PER_W // IDX_ROW
ROWS_PER_CHUNK = 4
NCHUNK = ROWS_PER_W // ROWS_PER_CHUNK
CHUNK = ROWS_PER_CHUNK * IDX_ROW


def _tr_body(in_ref, out_ref):
    h = in_ref[...].reshape(4 * D, XB)
    out_ref[...] = h.T


def _tc_transpose_group(table_t, fb):
    return pl.pallas_call(
        _tr_body,
        grid=(NXB,),
        in_specs=[pl.BlockSpec((4, D, XB), lambda b: (fb, 0, b))],
        out_specs=pl.BlockSpec((XB, 128), lambda b: (b, 0)),
        out_shape=jax.ShapeDtypeStruct((NXB * XB, 128), jnp.float32),
        name=f"transpose_g{fb}",
    )(table_t)


def _gather_body(fb, table_hbm, idx_hbm, out_hbm, idx_v, rows_v, sem):
    c = lax.axis_index("c")
    s = lax.axis_index("s")
    wid = s * 2 + c
    pltpu.sync_copy(
        idx_hbm.at[pl.ds(fb * (NW * ROWS_PER_W) + wid * ROWS_PER_W,
                         ROWS_PER_W)],
        idx_v)
    base = wid * PER_W

    def chunk_body(ci, _):
        r0 = ci * ROWS_PER_CHUNK
        for j in range(ROWS_PER_CHUNK):
            pltpu.async_copy(
                table_hbm.at[idx_v.at[r0 + j]],
                rows_v.at[pl.ds(j * IDX_ROW, IDX_ROW)],
                sem,
            )
        for j in range(ROWS_PER_CHUNK):
            pltpu.make_async_copy(
                table_hbm.at[idx_v.at[r0 + j]],
                rows_v.at[pl.ds(j * IDX_ROW, IDX_ROW)],
                sem,
            ).wait()
        pltpu.sync_copy(rows_v, out_hbm.at[pl.ds(base + ci * CHUNK, CHUNK)])
        return 0

    lax.fori_loop(0, NCHUNK, chunk_body, 0)


def _sc_gather(table_rows, idx_all, fb):
    mesh = plsc.VectorSubcoreMesh(core_axis_name="c", subcore_axis_name="s")
    return pl.kernel(
        functools.partial(_gather_body, fb),
        out_type=jax.ShapeDtypeStruct((B * 4, D), jnp.float32),
        mesh=mesh,
        scratch_types=[
            pltpu.VMEM((ROWS_PER_W, IDX_ROW), jnp.int32),
            pltpu.VMEM((CHUNK, D), jnp.float32),
            pltpu.SemaphoreType.DMA,
        ],
        compiler_params=pltpu.CompilerParams(use_tc_tiling_on_sc=False),
    )(table_rows, idx_all)


def _mlp_body(e0, e1, e2, e3, e4, e5, e6, xct_ref, w0, w1, w2, w3, w4, w5,
              w6, w1b_ref, b1_ref, w2_ref, b2_ref, w3_ref, b3_ref, out_ref):
    embs = (e0, e1, e2, e3, e4, e5, e6)
    ws = (w0, w1, w2, w3, w4, w5, w6)
    h = jnp.dot(embs[0][...], ws[0][...], preferred_element_type=jnp.float32)
    for i in range(1, FB):
        h = h + jnp.dot(embs[i][...], ws[i][...],
                        preferred_element_type=jnp.float32)
    h = h + lax.dot_general(xct_ref[...], w1b_ref[...],
                            (((0,), (0,)), ((), ())),
                            preferred_element_type=jnp.float32)
    h = jnp.maximum(h + b1_ref[...], 0.0)
    h = jnp.dot(h, w2_ref[...], preferred_element_type=jnp.float32)
    h = jnp.maximum(h + b2_ref[...], 0.0)
    o = jnp.dot(h, w3_ref[...], preferred_element_type=jnp.float32)
    out_ref[...] = o + b3_ref[...]


BM = 1024


def _mlp(embs, xct, w1g, w1b, b1, w2p, b2p, w3p, b3p):
    grid = (B // BM,)
    emb_specs = [pl.BlockSpec((BM, 128), lambda i: (i, 0)) for _ in range(FB)]
    w_specs = [pl.BlockSpec((128, H1), lambda i: (0, 0)) for _ in range(FB)]
    return pl.pallas_call(
        _mlp_body,
        grid=grid,
        in_specs=emb_specs + [pl.BlockSpec((C, BM), lambda i: (0, i))]
        + w_specs + [
            pl.BlockSpec((C, H1), lambda i: (0, 0)),
            pl.BlockSpec((1, H1), lambda i: (0, 0)),
            pl.BlockSpec((H1, 128), lambda i: (0, 0)),
            pl.BlockSpec((1, 128), lambda i: (0, 0)),
            pl.BlockSpec((128, 128), lambda i: (0, 0)),
            pl.BlockSpec((1, 128), lambda i: (0, 0)),
        ],
        out_specs=pl.BlockSpec((BM, 128), lambda i: (i, 0)),
        out_shape=jax.ShapeDtypeStruct((B, 128), jnp.float32),
    )(*embs, xct, *w1g, w1b, b1, w2p, b2p, w3p, b3p)


@jax.jit
def kernel(x_cat, x_cont, tables, W1, b1, W2, b2, W3, b3):
    table_t = tables.transpose(0, 2, 1)

    x = x_cat.astype(jnp.int32)
    fk = (jnp.arange(F, dtype=jnp.int32) % 4)[None, :]
    loc_idx = (x // XB) * (XB * 4) + (x % XB) * 4 + fk
    dummy = (jnp.arange(B, dtype=jnp.int32)[:, None] * 2
             + jnp.arange(2, dtype=jnp.int32)[None, :])
    loc28 = jnp.concatenate([loc_idx, dummy], axis=1)
    idx_all = (loc28.reshape(B, FB, 4).transpose(1, 0, 2)
               .reshape(FB * (B * 4) // IDX_ROW, IDX_ROW))

    embs = []
    for fb in range(FB):
        slab = _tc_transpose_group(table_t, fb)
        rows = slab.reshape(SLAB, D)
        embs.append(_sc_gather(rows, idx_all, fb).reshape(B, 128))

    xct = x_cont.T
    w1a = jnp.pad(W1[:F * D], ((0, (FB * 4 - F) * D), (0, 0)))
    w1g = [w1a[fb * 128:(fb + 1) * 128] for fb in range(FB)]
    w1b = W1[F * D:]
    w2p = jnp.pad(W2, ((0, 0), (0, 128 - H2)))
    b2p = jnp.pad(b2, (0, 128 - H2)).reshape(1, 128)
    w3p = jnp.pad(W3, ((0, 128 - H2), (0, 128 - NCLS)))
    b3p = jnp.pad(b3, (0, 128 - NCLS)).reshape(1, 128)

    out = _mlp(embs, xct, w1g, w1b, b1.reshape(1, H1), w2p, b2p, w3p, b3p)
    return out[:, :NCLS]


# speedup vs baseline: 2.0255x; 1.1582x over previous
import functools

import jax
import jax.numpy as jnp
from jax import lax
from jax.experimental import pallas as pl
from jax.experimental.pallas import tpu as pltpu
from jax.experimental.pallas import tpu_sc as plsc

B = 16384
F = 26
V = 100000
D = 32
C = 13
H1 = 128
H2 = 64
NCLS = 2

XB = 2048
NXB = 49
VPAD = NXB * XB
FB = (F + 3) // 4
SLAB = NXB * XB * 4

NW = 32
PER_W = (B * 4) // NW
IDX_ROW = 128
ROWS_PER_W = PER_W // IDX_ROW
ROWS_PER_CHUNK = 4
NCHUNK = ROWS_PER_W // ROWS_PER_CHUNK
CHUNK = ROWS_PER_CHUNK * IDX_ROW


def _tr_body(in_ref, out_ref):
    h = in_ref[...].reshape(4 * D, XB)
    out_ref[...] = h.T


def _tc_transpose_group(table_t, fb):
    return pl.pallas_call(
        _tr_body,
        grid=(NXB,),
        in_specs=[pl.BlockSpec((4, D, XB), lambda b: (fb, 0, b))],
        out_specs=pl.BlockSpec((XB, 128), lambda b: (b, 0)),
        out_shape=jax.ShapeDtypeStruct((NXB * XB, 128), jnp.float32),
        name=f"transpose_g{fb}",
    )(table_t)


def _gather_body(fb, table_hbm, idxt_hbm, out_hbm, idx_stage, idx_v, rows_v,
                 sem):
    c = lax.axis_index("c")
    s = lax.axis_index("s")
    wid = s * 2 + c
    nb = PER_W // 4
    pltpu.sync_copy(
        idxt_hbm.at[pl.ds(fb * 4, 4), pl.ds(wid * nb, nb)], idx_stage)
    lanes4 = lax.iota(jnp.int32, 16) * 4
    for g in range(nb // 16):
        for j in range(4):
            v = idx_stage[j, pl.ds(g * 16, 16)]
            plsc.store_scatter(idx_v, [lanes4 + (g * 64 + j)], v)
    base = wid * PER_W

    def chunk_body(ci, _):
        r0 = ci * ROWS_PER_CHUNK
        for j in range(ROWS_PER_CHUNK):
            pltpu.async_copy(
                table_hbm.at[idx_v.at[pl.ds((r0 + j) * IDX_ROW, IDX_ROW)]],
                rows_v.at[pl.ds(j * IDX_ROW, IDX_ROW)],
                sem,
            )
        for j in range(ROWS_PER_CHUNK):
            pltpu.make_async_copy(
                table_hbm.at[idx_v.at[pl.ds((r0 + j) * IDX_ROW, IDX_ROW)]],
                rows_v.at[pl.ds(j * IDX_ROW, IDX_ROW)],
                sem,
            ).wait()
        pltpu.sync_copy(rows_v, out_hbm.at[pl.ds(base + ci * CHUNK, CHUNK)])
        return 0

    lax.fori_loop(0, NCHUNK, chunk_body, 0)


def _sc_gather(table_rows, idx_all, fb):
    mesh = plsc.VectorSubcoreMesh(core_axis_name="c", subcore_axis_name="s")
    return pl.kernel(
        functools.partial(_gather_body, fb),
        out_type=jax.ShapeDtypeStruct((B * 4, D), jnp.float32),
        mesh=mesh,
        scratch_types=[
            pltpu.VMEM((4, PER_W // 4), jnp.int32),
            pltpu.VMEM((PER_W,), jnp.int32),
            pltpu.VMEM((CHUNK, D), jnp.float32),
            pltpu.SemaphoreType.DMA,
        ],
        compiler_params=pltpu.CompilerParams(
            use_tc_tiling_on_sc=False, needs_layout_passes=False),
    )(table_rows, idx_all)


def _mlp_body(e0, e1, e2, e3, e4, e5, e6, xct_ref, w0, w1, w2, w3, w4, w5,
              w6, w1b_ref, b1_ref, w2_ref, b2_ref, w3_ref, b3_ref, out_ref):
    embs = (e0, e1, e2, e3, e4, e5, e6)
    ws = (w0, w1, w2, w3, w4, w5, w6)
    h = jnp.dot(embs[0][...], ws[0][...], preferred_element_type=jnp.float32)
    for i in range(1, FB):
        h = h + jnp.dot(embs[i][...], ws[i][...],
                        preferred_element_type=jnp.float32)
    h = h + lax.dot_general(xct_ref[...], w1b_ref[...],
                            (((0,), (0,)), ((), ())),
                            preferred_element_type=jnp.float32)
    h = jnp.maximum(h + b1_ref[...], 0.0)
    h = jnp.dot(h, w2_ref[...], preferred_element_type=jnp.float32)
    h = jnp.maximum(h + b2_ref[...], 0.0)
    o = jnp.dot(h, w3_ref[...], preferred_element_type=jnp.float32)
    out_ref[...] = o + b3_ref[...]


BM = 1024


def _mlp(embs, xct, w1g, w1b, b1, w2p, b2p, w3p, b3p):
    grid = (B // BM,)
    emb_specs = [pl.BlockSpec((BM, 128), lambda i: (i, 0)) for _ in range(FB)]
    w_specs = [pl.BlockSpec((128, H1), lambda i: (0, 0)) for _ in range(FB)]
    return pl.pallas_call(
        _mlp_body,
        grid=grid,
        in_specs=emb_specs + [pl.BlockSpec((C, BM), lambda i: (0, i))]
        + w_specs + [
            pl.BlockSpec((C, H1), lambda i: (0, 0)),
            pl.BlockSpec((1, H1), lambda i: (0, 0)),
            pl.BlockSpec((H1, 128), lambda i: (0, 0)),
            pl.BlockSpec((1, 128), lambda i: (0, 0)),
            pl.BlockSpec((128, 128), lambda i: (0, 0)),
            pl.BlockSpec((1, 128), lambda i: (0, 0)),
        ],
        out_specs=pl.BlockSpec((BM, 128), lambda i: (i, 0)),
        out_shape=jax.ShapeDtypeStruct((B, 128), jnp.float32),
    )(*embs, xct, *w1g, w1b, b1, w2p, b2p, w3p, b3p)


@jax.jit
def kernel(x_cat, x_cont, tables, W1, b1, W2, b2, W3, b3):
    table_t = tables.transpose(0, 2, 1)

    xt = x_cat.T.astype(jnp.int32)
    fk = (jnp.arange(F, dtype=jnp.int32) % 4)[:, None]
    loct = (xt // XB) * (XB * 4) + (xt % XB) * 4 + fk
    dummy = (jnp.arange(B, dtype=jnp.int32)[None, :] * 2
             + jnp.arange(2, dtype=jnp.int32)[:, None])
    idx_all = jnp.concatenate([loct, dummy], axis=0)

    embs = []
    for fb in range(FB):
        slab = _tc_transpose_group(table_t, fb)
        rows = slab.reshape(SLAB, D)
        embs.append(_sc_gather(rows, idx_all, fb).reshape(B, 128))

    xct = x_cont.T
    w1a = jnp.pad(W1[:F * D], ((0, (FB * 4 - F) * D), (0, 0)))
    w1g = [w1a[fb * 128:(fb + 1) * 128] for fb in range(FB)]
    w1b = W1[F * D:]
    w2p = jnp.pad(W2, ((0, 0), (0, 128 - H2)))
    b2p = jnp.pad(b2, (0, 128 - H2)).reshape(1, 128)
    w3p = jnp.pad(W3, ((0, 128 - H2), (0, 128 - NCLS)))
    b3p = jnp.pad(b3, (0, 128 - NCLS)).reshape(1, 128)

    out = _mlp(embs, xct, w1g, w1b, b1.reshape(1, H1), w2p, b2p, w3p, b3p)
    return out[:, :NCLS]


# speedup vs baseline: 2.4986x; 1.2336x over previous
import functools

import jax
import jax.numpy as jnp
from jax import lax
from jax.experimental import pallas as pl
from jax.experimental.pallas import tpu as pltpu
from jax.experimental.pallas import tpu_sc as plsc

B = 16384
F = 26
V = 100000
D = 32
C = 13
H1 = 128
H2 = 64
NCLS = 2

XB = 4096
NXB = 25
VPAD = NXB * XB
FB = (F + 3) // 4
SLAB = NXB * XB * 4

NW = 32
PER_W = (B * 4) // NW
IDX_ROW = 128
ROWS_PER_W = PER_W // IDX_ROW
ROWS_PER_CHUNK = 4
NCHUNK = ROWS_PER_W // ROWS_PER_CHUNK
CHUNK = ROWS_PER_CHUNK * IDX_ROW


def _tr_body(in_ref, out_ref):
    h = in_ref[...].reshape(4 * D, XB)
    out_ref[...] = h.T


def _tc_transpose_group(table_t, fb):
    return pl.pallas_call(
        _tr_body,
        grid=(NXB,),
        in_specs=[pl.BlockSpec((4, D, XB), lambda b: (fb, 0, b))],
        out_specs=pl.BlockSpec((XB, 128), lambda b: (b, 0)),
        out_shape=jax.ShapeDtypeStruct((NXB * XB, 128), jnp.float32),
        name=f"transpose_g{fb}",
    )(table_t)


def _gather_body(fb, table_hbm, idxt_hbm, out_hbm, idx_stage, idx_v, rows_v,
                 sem):
    c = lax.axis_index("c")
    s = lax.axis_index("s")
    wid = s * 2 + c
    nb = PER_W // 4
    pltpu.sync_copy(
        idxt_hbm.at[pl.ds(fb * 4, 4), pl.ds(wid * nb, nb)], idx_stage)
    lanes4 = lax.iota(jnp.int32, 16) * 4
    for g in range(nb // 16):
        for j in range(4):
            v = idx_stage[j, pl.ds(g * 16, 16)]
            plsc.store_scatter(idx_v, [lanes4 + (g * 64 + j)], v)
    base = wid * PER_W

    def chunk_body(ci, _):
        r0 = ci * ROWS_PER_CHUNK
        for j in range(ROWS_PER_CHUNK):
            pltpu.async_copy(
                table_hbm.at[idx_v.at[pl.ds((r0 + j) * IDX_ROW, IDX_ROW)]],
                rows_v.at[pl.ds(j * IDX_ROW, IDX_ROW)],
                sem,
            )
        for j in range(ROWS_PER_CHUNK):
            pltpu.make_async_copy(
                table_hbm.at[idx_v.at[pl.ds((r0 + j) * IDX_ROW, IDX_ROW)]],
                rows_v.at[pl.ds(j * IDX_ROW, IDX_ROW)],
                sem,
            ).wait()
        pltpu.sync_copy(rows_v, out_hbm.at[pl.ds(base + ci * CHUNK, CHUNK)])
        return 0

    lax.fori_loop(0, NCHUNK, chunk_body, 0)


def _sc_gather(table_rows, idx_all, fb):
    mesh = plsc.VectorSubcoreMesh(core_axis_name="c", subcore_axis_name="s")
    return pl.kernel(
        functools.partial(_gather_body, fb),
        out_type=jax.ShapeDtypeStruct((B * 4, D), jnp.float32),
        mesh=mesh,
        scratch_types=[
            pltpu.VMEM((4, PER_W // 4), jnp.int32),
            pltpu.VMEM((PER_W,), jnp.int32),
            pltpu.VMEM((CHUNK, D), jnp.float32),
            pltpu.SemaphoreType.DMA,
        ],
        compiler_params=pltpu.CompilerParams(
            use_tc_tiling_on_sc=False, needs_layout_passes=False),
    )(table_rows, idx_all)


def _mlp_body(e0, e1, e2, e3, e4, e5, e6, xct_ref, w0, w1, w2, w3, w4, w5,
              w6, w1b_ref, b1_ref, w2_ref, b2_ref, w3_ref, b3_ref, out_ref):
    embs = (e0, e1, e2, e3, e4, e5, e6)
    ws = (w0, w1, w2, w3, w4, w5, w6)
    h = jnp.dot(embs[0][...], ws[0][...], preferred_element_type=jnp.float32)
    for i in range(1, FB):
        h = h + jnp.dot(embs[i][...], ws[i][...],
                        preferred_element_type=jnp.float32)
    h = h + lax.dot_general(xct_ref[...], w1b_ref[...],
                            (((0,), (0,)), ((), ())),
                            preferred_element_type=jnp.float32)
    h = jnp.maximum(h + b1_ref[...], 0.0)
    h = jnp.dot(h, w2_ref[...], preferred_element_type=jnp.float32)
    h = jnp.maximum(h + b2_ref[...], 0.0)
    o = jnp.dot(h, w3_ref[...], preferred_element_type=jnp.float32)
    out_ref[...] = o + b3_ref[...]


BM = 1024


def _mlp(embs, xct, w1g, w1b, b1, w2p, b2p, w3p, b3p):
    grid = (B // BM,)
    emb_specs = [pl.BlockSpec((BM, 128), lambda i: (i, 0)) for _ in range(FB)]
    w_specs = [pl.BlockSpec((128, H1), lambda i: (0, 0)) for _ in range(FB)]
    return pl.pallas_call(
        _mlp_body,
        grid=grid,
        in_specs=emb_specs + [pl.BlockSpec((C, BM), lambda i: (0, i))]
        + w_specs + [
            pl.BlockSpec((C, H1), lambda i: (0, 0)),
            pl.BlockSpec((1, H1), lambda i: (0, 0)),
            pl.BlockSpec((H1, 128), lambda i: (0, 0)),
            pl.BlockSpec((1, 128), lambda i: (0, 0)),
            pl.BlockSpec((128, 128), lambda i: (0, 0)),
            pl.BlockSpec((1, 128), lambda i: (0, 0)),
        ],
        out_specs=pl.BlockSpec((BM, 128), lambda i: (i, 0)),
        out_shape=jax.ShapeDtypeStruct((B, 128), jnp.float32),
    )(*embs, xct, *w1g, w1b, b1, w2p, b2p, w3p, b3p)


@jax.jit
def kernel(x_cat, x_cont, tables, W1, b1, W2, b2, W3, b3):
    table_t = tables.transpose(0, 2, 1)

    xt = x_cat.T.astype(jnp.int32)
    fk = (jnp.arange(F, dtype=jnp.int32) % 4)[:, None]
    loct = (xt // XB) * (XB * 4) + (xt % XB) * 4 + fk
    dummy = (jnp.arange(B, dtype=jnp.int32)[None, :] * 2
             + jnp.arange(2, dtype=jnp.int32)[:, None])
    idx_all = jnp.concatenate([loct, dummy], axis=0)

    embs = []
    for fb in range(FB):
        slab = _tc_transpose_group(table_t, fb)
        rows = slab.reshape(SLAB, D)
        embs.append(_sc_gather(rows, idx_all, fb).reshape(B, 128))

    xct = x_cont.T
    w1a = jnp.pad(W1[:F * D], ((0, (FB * 4 - F) * D), (0, 0)))
    w1g = [w1a[fb * 128:(fb + 1) * 128] for fb in range(FB)]
    w1b = W1[F * D:]
    w2p = jnp.pad(W2, ((0, 0), (0, 128 - H2)))
    b2p = jnp.pad(b2, (0, 128 - H2)).reshape(1, 128)
    w3p = jnp.pad(W3, ((0, 128 - H2), (0, 128 - NCLS)))
    b3p = jnp.pad(b3, (0, 128 - NCLS)).reshape(1, 128)

    out = _mlp(embs, xct, w1g, w1b, b1.reshape(1, H1), w2p, b2p, w3p, b3p)
    return out[:, :NCLS]


# speedup vs baseline: 2.6944x; 1.0784x over previous
import functools

import jax
import jax.numpy as jnp
from jax import lax
from jax.experimental import pallas as pl
from jax.experimental.pallas import tpu as pltpu
from jax.experimental.pallas import tpu_sc as plsc

B = 16384
F = 26
V = 100000
D = 32
C = 13
H1 = 128
H2 = 64
NCLS = 2

XB = 8192
NXB = 13
VPAD = NXB * XB
FB = (F + 3) // 4
SLAB = NXB * XB * 4

NW = 32
PER_W = (B * 4) // NW
IDX_ROW = 128
ROWS_PER_W = PER_W // IDX_ROW
ROWS_PER_CHUNK = 4
NCHUNK = ROWS_PER_W // ROWS_PER_CHUNK
CHUNK = ROWS_PER_CHUNK * IDX_ROW


def _tr_body(in_ref, out_ref):
    h = in_ref[...].reshape(4 * D, XB)
    out_ref[...] = h.T


def _tc_transpose_group(table_t, fb):
    return pl.pallas_call(
        _tr_body,
        grid=(NXB,),
        in_specs=[pl.BlockSpec((4, D, XB), lambda b: (fb, 0, b))],
        out_specs=pl.BlockSpec((XB, 128), lambda b: (b, 0)),
        out_shape=jax.ShapeDtypeStruct((NXB * XB, 128), jnp.float32),
        name=f"transpose_g{fb}",
    )(table_t)


def _gather_body(fb, table_hbm, idxt_hbm, out_hbm, idx_stage, idx_v, rows_v,
                 sem):
    c = lax.axis_index("c")
    s = lax.axis_index("s")
    wid = s * 2 + c
    nb = PER_W // 4
    pltpu.sync_copy(
        idxt_hbm.at[pl.ds(fb * 4, 4), pl.ds(wid * nb, nb)], idx_stage)
    lanes4 = lax.iota(jnp.int32, 16) * 4
    for g in range(nb // 16):
        for j in range(4):
            v = idx_stage[j, pl.ds(g * 16, 16)]
            plsc.store_scatter(idx_v, [lanes4 + (g * 64 + j)], v)
    base = wid * PER_W

    def chunk_body(ci, _):
        r0 = ci * ROWS_PER_CHUNK
        for j in range(ROWS_PER_CHUNK):
            pltpu.async_copy(
                table_hbm.at[idx_v.at[pl.ds((r0 + j) * IDX_ROW, IDX_ROW)]],
                rows_v.at[pl.ds(j * IDX_ROW, IDX_ROW)],
                sem,
            )
        for j in range(ROWS_PER_CHUNK):
            pltpu.make_async_copy(
                table_hbm.at[idx_v.at[pl.ds((r0 + j) * IDX_ROW, IDX_ROW)]],
                rows_v.at[pl.ds(j * IDX_ROW, IDX_ROW)],
                sem,
            ).wait()
        pltpu.sync_copy(rows_v, out_hbm.at[pl.ds(base + ci * CHUNK, CHUNK)])
        return 0

    lax.fori_loop(0, NCHUNK, chunk_body, 0)


def _sc_gather(table_rows, idx_all, fb):
    mesh = plsc.VectorSubcoreMesh(core_axis_name="c", subcore_axis_name="s")
    return pl.kernel(
        functools.partial(_gather_body, fb),
        out_type=jax.ShapeDtypeStruct((B * 4, D), jnp.float32),
        mesh=mesh,
        scratch_types=[
            pltpu.VMEM((4, PER_W // 4), jnp.int32),
            pltpu.VMEM((PER_W,), jnp.int32),
            pltpu.VMEM((CHUNK, D), jnp.float32),
            pltpu.SemaphoreType.DMA,
        ],
        compiler_params=pltpu.CompilerParams(
            use_tc_tiling_on_sc=False, needs_layout_passes=False),
    )(table_rows, idx_all)


def _mlp_body(e0, e1, e2, e3, e4, e5, e6, xct_ref, w0, w1, w2, w3, w4, w5,
              w6, w1b_ref, b1_ref, w2_ref, b2_ref, w3_ref, b3_ref, out_ref):
    embs = (e0, e1, e2, e3, e4, e5, e6)
    ws = (w0, w1, w2, w3, w4, w5, w6)
    h = jnp.dot(embs[0][...], ws[0][...], preferred_element_type=jnp.float32)
    for i in range(1, FB):
        h = h + jnp.dot(embs[i][...], ws[i][...],
                        preferred_element_type=jnp.float32)
    h = h + lax.dot_general(xct_ref[...], w1b_ref[...],
                            (((0,), (0,)), ((), ())),
                            preferred_element_type=jnp.float32)
    h = jnp.maximum(h + b1_ref[...], 0.0)
    h = jnp.dot(h, w2_ref[...], preferred_element_type=jnp.float32)
    h = jnp.maximum(h + b2_ref[...], 0.0)
    o = jnp.dot(h, w3_ref[...], preferred_element_type=jnp.float32)
    out_ref[...] = o + b3_ref[...]


BM = 1024


def _mlp(embs, xct, w1g, w1b, b1, w2p, b2p, w3p, b3p):
    grid = (B // BM,)
    emb_specs = [pl.BlockSpec((BM, 128), lambda i: (i, 0)) for _ in range(FB)]
    w_specs = [pl.BlockSpec((128, H1), lambda i: (0, 0)) for _ in range(FB)]
    return pl.pallas_call(
        _mlp_body,
        grid=grid,
        in_specs=emb_specs + [pl.BlockSpec((C, BM), lambda i: (0, i))]
        + w_specs + [
            pl.BlockSpec((C, H1), lambda i: (0, 0)),
            pl.BlockSpec((1, H1), lambda i: (0, 0)),
            pl.BlockSpec((H1, 128), lambda i: (0, 0)),
            pl.BlockSpec((1, 128), lambda i: (0, 0)),
            pl.BlockSpec((128, 128), lambda i: (0, 0)),
            pl.BlockSpec((1, 128), lambda i: (0, 0)),
        ],
        out_specs=pl.BlockSpec((BM, 128), lambda i: (i, 0)),
        out_shape=jax.ShapeDtypeStruct((B, 128), jnp.float32),
    )(*embs, xct, *w1g, w1b, b1, w2p, b2p, w3p, b3p)


@jax.jit
def kernel(x_cat, x_cont, tables, W1, b1, W2, b2, W3, b3):
    table_t = tables.transpose(0, 2, 1)

    xt = x_cat.T.astype(jnp.int32)
    fk = (jnp.arange(F, dtype=jnp.int32) % 4)[:, None]
    loct = (xt // XB) * (XB * 4) + (xt % XB) * 4 + fk
    dummy = (jnp.arange(B, dtype=jnp.int32)[None, :] * 2
             + jnp.arange(2, dtype=jnp.int32)[:, None])
    idx_all = jnp.concatenate([loct, dummy], axis=0)

    embs = []
    for fb in range(FB):
        slab = _tc_transpose_group(table_t, fb)
        rows = slab.reshape(SLAB, D)
        embs.append(_sc_gather(rows, idx_all, fb).reshape(B, 128))

    xct = x_cont.T
    w1a = jnp.pad(W1[:F * D], ((0, (FB * 4 - F) * D), (0, 0)))
    w1g = [w1a[fb * 128:(fb + 1) * 128] for fb in range(FB)]
    w1b = W1[F * D:]
    w2p = jnp.pad(W2, ((0, 0), (0, 128 - H2)))
    b2p = jnp.pad(b2, (0, 128 - H2)).reshape(1, 128)
    w3p = jnp.pad(W3, ((0, 128 - H2), (0, 128 - NCLS)))
    b3p = jnp.pad(b3, (0, 128 - NCLS)).reshape(1, 128)

    out = _mlp(embs, xct, w1g, w1b, b1.reshape(1, H1), w2p, b2p, w3p, b3p)
    return out[:, :NCLS]


# speedup vs baseline: 2.7029x; 1.0031x over previous
import functools

import jax
import jax.numpy as jnp
from jax import lax
from jax.experimental import pallas as pl
from jax.experimental.pallas import tpu as pltpu
from jax.experimental.pallas import tpu_sc as plsc

B = 16384
F = 26
V = 100000
D = 32
C = 13
H1 = 128
H2 = 64
NCLS = 2

XB = 16384
NXB = 7
VPAD = NXB * XB
FB = (F + 3) // 4
SLAB = NXB * XB * 4

NW = 32
PER_W = (B * 4) // NW
IDX_ROW = 128
ROWS_PER_W = PER_W // IDX_ROW
ROWS_PER_CHUNK = 4
NCHUNK = ROWS_PER_W // ROWS_PER_CHUNK
CHUNK = ROWS_PER_CHUNK * IDX_ROW


def _tr_body(in_ref, out_ref):
    h = in_ref[...].reshape(4 * D, XB)
    out_ref[...] = h.T


def _tc_transpose_group(table_t, fb):
    return pl.pallas_call(
        _tr_body,
        grid=(NXB,),
        in_specs=[pl.BlockSpec((4, D, XB), lambda b: (fb, 0, b))],
        out_specs=pl.BlockSpec((XB, 128), lambda b: (b, 0)),
        out_shape=jax.ShapeDtypeStruct((NXB * XB, 128), jnp.float32),
        name=f"transpose_g{fb}",
    )(table_t)


def _gather_body(fb, table_hbm, idxt_hbm, out_hbm, idx_stage, idx_v, rows_v,
                 sem):
    c = lax.axis_index("c")
    s = lax.axis_index("s")
    wid = s * 2 + c
    nb = PER_W // 4
    pltpu.sync_copy(
        idxt_hbm.at[pl.ds(fb * 4, 4), pl.ds(wid * nb, nb)], idx_stage)
    lanes4 = lax.iota(jnp.int32, 16) * 4
    for g in range(nb // 16):
        for j in range(4):
            v = idx_stage[j, pl.ds(g * 16, 16)]
            plsc.store_scatter(idx_v, [lanes4 + (g * 64 + j)], v)
    base = wid * PER_W

    def chunk_body(ci, _):
        r0 = ci * ROWS_PER_CHUNK
        for j in range(ROWS_PER_CHUNK):
            pltpu.async_copy(
                table_hbm.at[idx_v.at[pl.ds((r0 + j) * IDX_ROW, IDX_ROW)]],
                rows_v.at[pl.ds(j * IDX_ROW, IDX_ROW)],
                sem,
            )
        for j in range(ROWS_PER_CHUNK):
            pltpu.make_async_copy(
                table_hbm.at[idx_v.at[pl.ds((r0 + j) * IDX_ROW, IDX_ROW)]],
                rows_v.at[pl.ds(j * IDX_ROW, IDX_ROW)],
                sem,
            ).wait()
        pltpu.sync_copy(rows_v, out_hbm.at[pl.ds(base + ci * CHUNK, CHUNK)])
        return 0

    lax.fori_loop(0, NCHUNK, chunk_body, 0)


def _sc_gather(table_rows, idx_all, fb):
    mesh = plsc.VectorSubcoreMesh(core_axis_name="c", subcore_axis_name="s")
    return pl.kernel(
        functools.partial(_gather_body, fb),
        out_type=jax.ShapeDtypeStruct((B * 4, D), jnp.float32),
        mesh=mesh,
        scratch_types=[
            pltpu.VMEM((4, PER_W // 4), jnp.int32),
            pltpu.VMEM((PER_W,), jnp.int32),
            pltpu.VMEM((CHUNK, D), jnp.float32),
            pltpu.SemaphoreType.DMA,
        ],
        compiler_params=pltpu.CompilerParams(
            use_tc_tiling_on_sc=False, needs_layout_passes=False),
    )(table_rows, idx_all)


def _mlp_body(e0, e1, e2, e3, e4, e5, e6, xct_ref, w0, w1, w2, w3, w4, w5,
              w6, w1b_ref, b1_ref, w2_ref, b2_ref, w3_ref, b3_ref, out_ref):
    embs = (e0, e1, e2, e3, e4, e5, e6)
    ws = (w0, w1, w2, w3, w4, w5, w6)
    h = jnp.dot(embs[0][...], ws[0][...], preferred_element_type=jnp.float32)
    for i in range(1, FB):
        h = h + jnp.dot(embs[i][...], ws[i][...],
                        preferred_element_type=jnp.float32)
    h = h + lax.dot_general(xct_ref[...], w1b_ref[...],
                            (((0,), (0,)), ((), ())),
                            preferred_element_type=jnp.float32)
    h = jnp.maximum(h + b1_ref[...], 0.0)
    h = jnp.dot(h, w2_ref[...], preferred_element_type=jnp.float32)
    h = jnp.maximum(h + b2_ref[...], 0.0)
    o = jnp.dot(h, w3_ref[...], preferred_element_type=jnp.float32)
    out_ref[...] = o + b3_ref[...]


BM = 1024


def _mlp(embs, xct, w1g, w1b, b1, w2p, b2p, w3p, b3p):
    grid = (B // BM,)
    emb_specs = [pl.BlockSpec((BM, 128), lambda i: (i, 0)) for _ in range(FB)]
    w_specs = [pl.BlockSpec((128, H1), lambda i: (0, 0)) for _ in range(FB)]
    return pl.pallas_call(
        _mlp_body,
        grid=grid,
        in_specs=emb_specs + [pl.BlockSpec((C, BM), lambda i: (0, i))]
        + w_specs + [
            pl.BlockSpec((C, H1), lambda i: (0, 0)),
            pl.BlockSpec((1, H1), lambda i: (0, 0)),
            pl.BlockSpec((H1, 128), lambda i: (0, 0)),
            pl.BlockSpec((1, 128), lambda i: (0, 0)),
            pl.BlockSpec((128, 128), lambda i: (0, 0)),
            pl.BlockSpec((1, 128), lambda i: (0, 0)),
        ],
        out_specs=pl.BlockSpec((BM, 128), lambda i: (i, 0)),
        out_shape=jax.ShapeDtypeStruct((B, 128), jnp.float32),
    )(*embs, xct, *w1g, w1b, b1, w2p, b2p, w3p, b3p)


@jax.jit
def kernel(x_cat, x_cont, tables, W1, b1, W2, b2, W3, b3):
    table_t = tables.transpose(0, 2, 1)

    xt = x_cat.T.astype(jnp.int32)
    fk = (jnp.arange(F, dtype=jnp.int32) % 4)[:, None]
    loct = (xt // XB) * (XB * 4) + (xt % XB) * 4 + fk
    dummy = (jnp.arange(B, dtype=jnp.int32)[None, :] * 2
             + jnp.arange(2, dtype=jnp.int32)[:, None])
    idx_all = jnp.concatenate([loct, dummy], axis=0)

    embs = []
    for fb in range(FB):
        slab = _tc_transpose_group(table_t, fb)
        rows = slab.reshape(SLAB, D)
        embs.append(_sc_gather(rows, idx_all, fb).reshape(B, 128))

    xct = x_cont.T
    w1a = jnp.pad(W1[:F * D], ((0, (FB * 4 - F) * D), (0, 0)))
    w1g = [w1a[fb * 128:(fb + 1) * 128] for fb in range(FB)]
    w1b = W1[F * D:]
    w2p = jnp.pad(W2, ((0, 0), (0, 128 - H2)))
    b2p = jnp.pad(b2, (0, 128 - H2)).reshape(1, 128)
    w3p = jnp.pad(W3, ((0, 128 - H2), (0, 128 - NCLS)))
    b3p = jnp.pad(b3, (0, 128 - NCLS)).reshape(1, 128)

    out = _mlp(embs, xct, w1g, w1b, b1.reshape(1, H1), w2p, b2p, w3p, b3p)
    return out[:, :NCLS]
